# baseline (speedup 1.0000x reference)
import jax
import jax.numpy as jnp
from jax.experimental import pallas as pl
from jax.experimental.pallas import tpu as pltpu

_BIT = 512
_SPIKE = 7.0

_ROWS = 384
_BLOCK = 16
_STEPS = _ROWS // _BLOCK
_NBUF = 6
_M = 224


def _quant(v):
    kf = jnp.clip(v / _SPIKE + 0.5, 0.0, float(_BIT - 1))
    k0 = kf.astype(jnp.int32)
    km = jnp.maximum(k0 - 1, 0)
    kp = jnp.minimum(k0 + 1, _BIT - 1)

    def dist(k):
        return jnp.abs(v - k.astype(jnp.float32) * _SPIKE)

    dm = dist(km)
    d0 = dist(k0)
    dp = dist(kp)
    best = km
    bd = dm
    t0 = d0 < bd
    best = jnp.where(t0, k0, best)
    bd = jnp.where(t0, d0, bd)
    tp = dp < bd
    best = jnp.where(tp, kp, best)
    return best.astype(jnp.float32) * _SPIKE


def _body(x_hbm, y_ref, o_hbm, xbuf, pat, rsems, wsem):
    def read(j, slot):
        pltpu.make_async_copy(
            x_hbm.at[pl.ds(j * _BLOCK, _BLOCK)],
            xbuf.at[slot],
            rsems.at[slot],
        ).start()

    def write(j):
        return pltpu.make_async_copy(
            pat, o_hbm.at[pl.ds(j * _BLOCK, _BLOCK)], wsem
        )

    for b in range(_NBUF):
        read(b, b)

    y = y_ref[...]
    ymax = jnp.max(y)
    ymin = jnp.min(y)
    v = y / (ymax - ymin) * _SPIKE * float(_BIT)
    vals = _quant(v)
    pat[...] = jnp.broadcast_to(vals[None], pat.shape)

    mn = None
    mx = None
    for j in range(_STEPS):
        slot = j % _NBUF
        pltpu.make_async_copy(
            x_hbm.at[pl.ds(j * _BLOCK, _BLOCK)],
            xbuf.at[slot],
            rsems.at[slot],
        ).wait()
        write(j).start()
        blk = xbuf[slot]
        bmn = jnp.min(blk)
        bmx = jnp.max(blk)
        mn = bmn if mn is None else jnp.minimum(mn, bmn)
        mx = bmx if mx is None else jnp.maximum(mx, bmx)
        if j + _NBUF < _STEPS:
            read(j + _NBUF, slot)

    for j in range(_STEPS):
        write(j).wait()

    n = (mx - mn).astype(jnp.int32) + 1

    @pl.when(n < _M)
    def _fixup():
        row = jax.lax.broadcasted_iota(jnp.int32, (_M, 1), 0)
        pat[...] = jnp.broadcast_to(
            jnp.where(row < n, vals, 0.0)[None], pat.shape
        )
        for j in range(_STEPS):
            write(j).start()
        for j in range(_STEPS):
            write(j).wait()


def kernel(x, y):
    out3 = pl.pallas_call(
        _body,
        in_specs=[
            pl.BlockSpec(memory_space=pl.ANY),
            pl.BlockSpec(memory_space=pltpu.VMEM),
        ],
        out_specs=pl.BlockSpec(memory_space=pl.ANY),
        out_shape=jax.ShapeDtypeStruct((_ROWS, 224, 224), jnp.float32),
        scratch_shapes=[
            pltpu.VMEM((_NBUF, _BLOCK, 224, 224), jnp.float32),
            pltpu.VMEM((_BLOCK, 224, 224), jnp.float32),
            pltpu.SemaphoreType.DMA((_NBUF,)),
            pltpu.SemaphoreType.DMA,
        ],
    )(x.reshape(_ROWS, 224, 224), y.reshape(_M, 1))
    return out3.reshape(x.shape)


# speedup vs baseline: 1.0047x; 1.0047x over previous
import jax
import jax.numpy as jnp
from jax.experimental import pallas as pl
from jax.experimental.pallas import tpu as pltpu

_BIT = 512
_SPIKE = 7.0

_ROWS = 384
_BLOCK = 24
_STEPS = _ROWS // _BLOCK
_NBUF = 4
_M = 224


def _quant(v):
    kf = jnp.clip(v / _SPIKE + 0.5, 0.0, float(_BIT - 1))
    k0 = kf.astype(jnp.int32)
    km = jnp.maximum(k0 - 1, 0)
    kp = jnp.minimum(k0 + 1, _BIT - 1)

    def dist(k):
        return jnp.abs(v - k.astype(jnp.float32) * _SPIKE)

    dm = dist(km)
    d0 = dist(k0)
    dp = dist(kp)
    best = km
    bd = dm
    t0 = d0 < bd
    best = jnp.where(t0, k0, best)
    bd = jnp.where(t0, d0, bd)
    tp = dp < bd
    best = jnp.where(tp, kp, best)
    return best.astype(jnp.float32) * _SPIKE


def _body(x_hbm, y_ref, o_hbm, xbuf, pat, rsems, wsem):
    def read(j, slot):
        pltpu.make_async_copy(
            x_hbm.at[pl.ds(j * _BLOCK, _BLOCK)],
            xbuf.at[slot],
            rsems.at[slot],
        ).start()

    def write(j):
        return pltpu.make_async_copy(
            pat, o_hbm.at[pl.ds(j * _BLOCK, _BLOCK)], wsem
        )

    for b in range(_NBUF):
        read(b, b)

    y = y_ref[...]
    ymax = jnp.max(y)
    ymin = jnp.min(y)
    v = y / (ymax - ymin) * _SPIKE * float(_BIT)
    vals = _quant(v)
    pat[...] = jnp.broadcast_to(vals[None], pat.shape)

    mn = None
    mx = None
    for j in range(_STEPS):
        slot = j % _NBUF
        pltpu.make_async_copy(
            x_hbm.at[pl.ds(j * _BLOCK, _BLOCK)],
            xbuf.at[slot],
            rsems.at[slot],
        ).wait()
        write(j).start()
        blk = xbuf[slot]
        bmn = jnp.min(blk)
        bmx = jnp.max(blk)
        mn = bmn if mn is None else jnp.minimum(mn, bmn)
        mx = bmx if mx is None else jnp.maximum(mx, bmx)
        if j + _NBUF < _STEPS:
            read(j + _NBUF, slot)

    for j in range(_STEPS):
        write(j).wait()

    n = (mx - mn).astype(jnp.int32) + 1

    @pl.when(n < _M)
    def _fixup():
        row = jax.lax.broadcasted_iota(jnp.int32, (_M, 1), 0)
        pat[...] = jnp.broadcast_to(
            jnp.where(row < n, vals, 0.0)[None], pat.shape
        )
        for j in range(_STEPS):
            write(j).start()
        for j in range(_STEPS):
            write(j).wait()


def kernel(x, y):
    out3 = pl.pallas_call(
        _body,
        in_specs=[
            pl.BlockSpec(memory_space=pl.ANY),
            pl.BlockSpec(memory_space=pltpu.VMEM),
        ],
        out_specs=pl.BlockSpec(memory_space=pl.ANY),
        out_shape=jax.ShapeDtypeStruct((_ROWS, 224, 224), jnp.float32),
        scratch_shapes=[
            pltpu.VMEM((_NBUF, _BLOCK, 224, 224), jnp.float32),
            pltpu.VMEM((_BLOCK, 224, 224), jnp.float32),
            pltpu.SemaphoreType.DMA((_NBUF,)),
            pltpu.SemaphoreType.DMA,
        ],
    )(x.reshape(_ROWS, 224, 224), y.reshape(_M, 1))
    return out3.reshape(x.shape)
